# baseline (speedup 1.0000x reference)
import functools

import jax
import jax.numpy as jnp
from jax import lax
from jax.experimental import pallas as pl
from jax.experimental.pallas import tpu as pltpu
from jax.experimental.pallas import tpu_sc as plsc

N = 10000
E = 320000
H = 128
G = 64
C = 10

NC = 2
NS = 16
NW = NC * NS
CHUNK = 128
NCHUNKS = E // CHUNK
FULL_TRIPS = NCHUNKS // NW
REM = NCHUNKS - FULL_TRIPS * NW
NPA = 10112
RPW = NPA // NS
NIB = 4
NRB = 2

_MESH = plsc.VectorSubcoreMesh(core_axis_name="c", subcore_axis_name="s")


@functools.partial(
    pl.kernel,
    out_type=jax.ShapeDtypeStruct((NC * NPA, H), jnp.float32),
    mesh=_MESH,
    scratch_types=[
        pltpu.VMEM((CHUNK,), jnp.int32),
        pltpu.VMEM((CHUNK, H), jnp.float32),
        pltpu.VMEM_SHARED((NPA, H), jnp.float32),
    ],
)
def _deg_kernel(dst_hbm, ones_hbm, zeros_hbm, out_hbm, didx, ones_v, acc):
    c = lax.axis_index("c")
    s = lax.axis_index("s")
    w = s * NC + c
    pltpu.sync_copy(ones_hbm, ones_v)
    arow = pl.multiple_of(s * RPW, 8)
    pltpu.sync_copy(zeros_hbm, acc.at[pl.ds(arow, RPW)])
    plsc.subcore_barrier()

    def trip(r):
        base = pl.multiple_of(r * CHUNK, 8)
        pltpu.sync_copy(dst_hbm.at[pl.ds(base, CHUNK)], didx)
        pltpu.sync_copy(ones_v, acc.at[didx], add=True)

    def body(j, carry):
        trip(w + j * NW)
        return carry

    lax.fori_loop(0, FULL_TRIPS, body, 0)

    @pl.when(w < REM)
    def _():
        trip(FULL_TRIPS * NW + w)

    plsc.subcore_barrier()
    orow = pl.multiple_of(c * NPA + s * RPW, 8)
    pltpu.sync_copy(acc.at[pl.ds(arow, RPW)], out_hbm.at[pl.ds(orow, RPW)])


@functools.partial(
    pl.kernel,
    out_type=jax.ShapeDtypeStruct((NC * NPA, H), jnp.float32),
    mesh=_MESH,
    scratch_types=(
        [pltpu.VMEM((CHUNK,), jnp.int32) for _ in range(NIB)]
        + [pltpu.VMEM((CHUNK,), jnp.int32) for _ in range(NIB)]
        + [pltpu.VMEM((CHUNK, H), jnp.float32) for _ in range(NRB)]
        + [pltpu.VMEM_SHARED((NPA, H), jnp.float32)]
        + [pltpu.SemaphoreType.DMA for _ in range(NIB + NRB)]
    ),
)
def _spmm_kernel(src_hbm, dst_hbm, g_hbm, zeros_hbm, out_hbm, *bufs):
    sib = bufs[0:NIB]
    dib = bufs[NIB:2 * NIB]
    rows = bufs[2 * NIB:2 * NIB + NRB]
    acc = bufs[2 * NIB + NRB]
    isem = bufs[2 * NIB + NRB + 1:2 * NIB + NRB + 1 + NIB]
    gsem = bufs[2 * NIB + NRB + 1 + NIB:]
    c = lax.axis_index("c")
    s = lax.axis_index("s")
    w = s * NC + c
    arow = pl.multiple_of(s * RPW, 8)
    pltpu.sync_copy(zeros_hbm, acc.at[pl.ds(arow, RPW)])
    plsc.subcore_barrier()

    def stage_idx(bi, cidx):
        base = pl.multiple_of((w + cidx * NW) * CHUNK, 8)
        pltpu.async_copy(src_hbm.at[pl.ds(base, CHUNK)], sib[bi], isem[bi])
        pltpu.async_copy(dst_hbm.at[pl.ds(base, CHUNK)], dib[bi], isem[bi])

    def wait_idx(bi):
        pltpu.make_async_copy(src_hbm.at[pl.ds(0, CHUNK)], sib[bi],
                              isem[bi]).wait()
        pltpu.make_async_copy(dst_hbm.at[pl.ds(0, CHUNK)], dib[bi],
                              isem[bi]).wait()

    def start_gather(bi, br):
        pltpu.async_copy(g_hbm.at[sib[bi]], rows[br], gsem[br])

    def wait_gather(bi, br):
        pltpu.make_async_copy(g_hbm.at[sib[bi]], rows[br], gsem[br]).wait()

    for bi in range(NIB):
        stage_idx(bi, bi)
    for bi in range(2):
        wait_idx(bi)
        start_gather(bi, bi)

    def group(t, carry):
        for b in range(NIB):
            cidx = t * NIB + b
            br = b % NRB
            wait_gather(b, br)
            pltpu.sync_copy(rows[br], acc.at[dib[b]], add=True)

            @pl.when(cidx + NIB < FULL_TRIPS)
            def _():
                stage_idx(b, cidx + NIB)

            @pl.when(cidx + 2 < FULL_TRIPS)
            def _():
                bi2 = (b + 2) % NIB
                wait_idx(bi2)
                start_gather(bi2, br)
        return carry

    lax.fori_loop(0, FULL_TRIPS // NIB, group, 0)

    for b in range(FULL_TRIPS % NIB):
        wait_gather(b, b % NRB)
        pltpu.sync_copy(rows[b % NRB], acc.at[dib[b]], add=True)

    @pl.when(w < REM)
    def _():
        base = pl.multiple_of((FULL_TRIPS * NW + w) * CHUNK, 8)
        pltpu.sync_copy(src_hbm.at[pl.ds(base, CHUNK)], sib[0])
        pltpu.sync_copy(dst_hbm.at[pl.ds(base, CHUNK)], dib[0])
        pltpu.async_copy(g_hbm.at[sib[0]], rows[0], gsem[0]).wait()
        pltpu.sync_copy(rows[0], acc.at[dib[0]], add=True)

    plsc.subcore_barrier()
    orow = pl.multiple_of(c * NPA + s * RPW, 8)
    pltpu.sync_copy(acc.at[pl.ds(arow, RPW)], out_hbm.at[pl.ds(orow, RPW)])


ROWS_BLK = 2000
NBLK = N // ROWS_BLK


def _tch_body(x, W1, h1_ref):
    h1_ref[...] = jnp.dot(x[...], W1[...],
                          preferred_element_type=jnp.float32)


def _tch(x, W1):
    return pl.pallas_call(
        _tch_body,
        grid=(NBLK,),
        in_specs=[
            pl.BlockSpec((ROWS_BLK, H), lambda i: (i, 0)),
            pl.BlockSpec((H, H), lambda i: (0, 0)),
        ],
        out_specs=pl.BlockSpec((ROWS_BLK, H), lambda i: (i, 0)),
        out_shape=jax.ShapeDtypeStruct((N, H), jnp.float32),
    )(x, W1)


def _tca_body(degp0, degp1, h1, dis_ref, g1_ref):
    deg = degp0[:, 0:1] + degp1[:, 0:1] + 1.0
    dis = lax.rsqrt(deg)
    dis_ref[...] = dis
    g1_ref[...] = h1[...] * dis


def _tca(degp0, degp1, h1):
    return pl.pallas_call(
        _tca_body,
        grid=(NBLK,),
        in_specs=[
            pl.BlockSpec((ROWS_BLK, H), lambda i: (i, 0)),
            pl.BlockSpec((ROWS_BLK, H), lambda i: (i, 0)),
            pl.BlockSpec((ROWS_BLK, H), lambda i: (i, 0)),
        ],
        out_specs=[
            pl.BlockSpec((ROWS_BLK, 1), lambda i: (i, 0)),
            pl.BlockSpec((ROWS_BLK, H), lambda i: (i, 0)),
        ],
        out_shape=[
            jax.ShapeDtypeStruct((N, 1), jnp.float32),
            jax.ShapeDtypeStruct((N, H), jnp.float32),
        ],
    )(degp0, degp1, h1)


def _tcb_body(sp0, sp1, g, dis, b, W, gn_ref):
    a = jnp.maximum(dis[...] * (sp0[...] + sp1[...] + g[...]) + b[...], 0.0)
    gn_ref[...] = dis[...] * jnp.dot(a, W[...],
                                     preferred_element_type=jnp.float32)


def _tcb(sp0, sp1, g, dis, b, W):
    return pl.pallas_call(
        _tcb_body,
        grid=(NBLK,),
        in_specs=[
            pl.BlockSpec((ROWS_BLK, H), lambda i: (i, 0)),
            pl.BlockSpec((ROWS_BLK, H), lambda i: (i, 0)),
            pl.BlockSpec((ROWS_BLK, H), lambda i: (i, 0)),
            pl.BlockSpec((ROWS_BLK, 1), lambda i: (i, 0)),
            pl.BlockSpec((1, H), lambda i: (0, 0)),
            pl.BlockSpec((H, H), lambda i: (0, 0)),
        ],
        out_specs=pl.BlockSpec((ROWS_BLK, H), lambda i: (i, 0)),
        out_shape=jax.ShapeDtypeStruct((N, H), jnp.float32),
    )(sp0, sp1, g, dis, b, W)


def _tcc_body(sp0, sp1, g, dis, b, batchb, Wl, bl, out_ref, sums, counts):
    i = pl.program_id(0)

    @pl.when(i == 0)
    def _():
        sums[...] = jnp.zeros_like(sums)
        counts[...] = jnp.zeros_like(counts)

    a = dis[...] * (sp0[...] + sp1[...] + g[...]) + b[...]
    gid = lax.broadcasted_iota(jnp.int32, (ROWS_BLK, G), 1)
    m = (batchb[...] == gid).astype(jnp.float32)
    sums[...] += lax.dot_general(m, a, (((0,), (0,)), ((), ())),
                                 preferred_element_type=jnp.float32)
    counts[...] += lax.dot_general(
        m, jnp.ones((ROWS_BLK, H), jnp.float32), (((0,), (0,)), ((), ())),
        preferred_element_type=jnp.float32)

    @pl.when(i == NBLK - 1)
    def _():
        pooled = sums[...] / jnp.maximum(counts[...], 1.0)
        logits = jnp.dot(pooled, Wl[...],
                         preferred_element_type=jnp.float32) + bl[...]
        z = logits - jnp.max(logits, axis=-1, keepdims=True)
        e = jnp.exp(z)
        out_ref[...] = e / jnp.sum(e, axis=-1, keepdims=True)


def _tcc(sp0, sp1, g, dis, b, batch2d, Wl, bl):
    return pl.pallas_call(
        _tcc_body,
        grid=(NBLK,),
        in_specs=[
            pl.BlockSpec((ROWS_BLK, H), lambda i: (i, 0)),
            pl.BlockSpec((ROWS_BLK, H), lambda i: (i, 0)),
            pl.BlockSpec((ROWS_BLK, H), lambda i: (i, 0)),
            pl.BlockSpec((ROWS_BLK, 1), lambda i: (i, 0)),
            pl.BlockSpec((1, H), lambda i: (0, 0)),
            pl.BlockSpec((ROWS_BLK, 1), lambda i: (i, 0)),
            pl.BlockSpec((H, C), lambda i: (0, 0)),
            pl.BlockSpec((1, C), lambda i: (0, 0)),
        ],
        out_specs=pl.BlockSpec((G, C), lambda i: (0, 0)),
        out_shape=jax.ShapeDtypeStruct((G, C), jnp.float32),
        scratch_shapes=[
            pltpu.VMEM((G, H), jnp.float32),
            pltpu.VMEM((G, H), jnp.float32),
        ],
    )(sp0, sp1, g, dis, b, batch2d, Wl, bl)


def kernel(x, edge_index, batch, W1, b1, W2, b2, W3, b3, Wl, bl):
    src1d = edge_index[0]
    dst1d = edge_index[1]
    batch2d = batch.reshape(N, 1)
    zeros_spmm = jnp.zeros((RPW, H), jnp.float32)
    ones_rows = jnp.ones((CHUNK, H), jnp.float32)
    b1r, b2r, b3r = b1.reshape(1, H), b2.reshape(1, H), b3.reshape(1, H)
    blr = bl.reshape(1, C)

    degp = _deg_kernel(dst1d, ones_rows, zeros_spmm)
    h1 = _tch(x, W1)
    dis, g1 = _tca(degp[:N], degp[NPA:NPA + N], h1)
    s1 = _spmm_kernel(src1d, dst1d, g1, zeros_spmm)
    g2 = _tcb(s1[:N], s1[NPA:NPA + N], g1, dis, b1r, W2)
    s2 = _spmm_kernel(src1d, dst1d, g2, zeros_spmm)
    g3 = _tcb(s2[:N], s2[NPA:NPA + N], g2, dis, b2r, W3)
    s3 = _spmm_kernel(src1d, dst1d, g3, zeros_spmm)
    return _tcc(s3[:N], s3[NPA:NPA + N], g3, dis, b3r, batch2d, Wl, blr)


# speedup vs baseline: 1.0037x; 1.0037x over previous
import functools

import jax
import jax.numpy as jnp
from jax import lax
from jax.experimental import pallas as pl
from jax.experimental.pallas import tpu as pltpu
from jax.experimental.pallas import tpu_sc as plsc

N = 10000
E = 320000
H = 128
G = 64
C = 10

NC = 2
NS = 16
NW = NC * NS
CHUNK = 128
NCHUNKS = E // CHUNK
FULL_TRIPS = NCHUNKS // NW
REM = NCHUNKS - FULL_TRIPS * NW
NPA = 10112
RPW = NPA // NS
NIB = 4
NRB = 2

_MESH = plsc.VectorSubcoreMesh(core_axis_name="c", subcore_axis_name="s")


@functools.partial(
    pl.kernel,
    out_type=jax.ShapeDtypeStruct((NC * NPA, H), jnp.float32),
    mesh=_MESH,
    scratch_types=[
        pltpu.VMEM((CHUNK,), jnp.int32),
        pltpu.VMEM((CHUNK, H), jnp.float32),
        pltpu.VMEM_SHARED((NPA, H), jnp.float32),
    ],
)
def _deg_kernel(dst_hbm, ones_hbm, zeros_hbm, out_hbm, didx, ones_v, acc):
    c = lax.axis_index("c")
    s = lax.axis_index("s")
    w = s * NC + c
    pltpu.sync_copy(ones_hbm, ones_v)
    arow = pl.multiple_of(s * RPW, 8)
    pltpu.sync_copy(zeros_hbm, acc.at[pl.ds(arow, RPW)])
    plsc.subcore_barrier()

    def trip(r):
        base = pl.multiple_of(r * CHUNK, 8)
        pltpu.sync_copy(dst_hbm.at[pl.ds(base, CHUNK)], didx)
        pltpu.sync_copy(ones_v, acc.at[didx], add=True)

    def body(j, carry):
        trip(w + j * NW)
        return carry

    lax.fori_loop(0, FULL_TRIPS, body, 0)

    @pl.when(w < REM)
    def _():
        trip(FULL_TRIPS * NW + w)

    plsc.subcore_barrier()
    orow = pl.multiple_of(c * NPA + s * RPW, 8)
    pltpu.sync_copy(acc.at[pl.ds(arow, RPW)], out_hbm.at[pl.ds(orow, RPW)])


@functools.partial(
    pl.kernel,
    out_type=jax.ShapeDtypeStruct((NC * NPA, H), jnp.float32),
    mesh=_MESH,
    scratch_types=(
        [pltpu.VMEM((CHUNK,), jnp.int32) for _ in range(NIB)]
        + [pltpu.VMEM((CHUNK,), jnp.int32) for _ in range(NIB)]
        + [pltpu.VMEM((CHUNK, H), jnp.float32) for _ in range(NRB)]
        + [pltpu.VMEM_SHARED((NPA, H), jnp.float32)]
        + [pltpu.SemaphoreType.DMA for _ in range(NIB + NRB)]
    ),
)
def _spmm_kernel(src_hbm, dst_hbm, g_hbm, zeros_hbm, out_hbm, *bufs):
    sib = bufs[0:NIB]
    dib = bufs[NIB:2 * NIB]
    rows = bufs[2 * NIB:2 * NIB + NRB]
    acc = bufs[2 * NIB + NRB]
    isem = bufs[2 * NIB + NRB + 1:2 * NIB + NRB + 1 + NIB]
    gsem = bufs[2 * NIB + NRB + 1 + NIB:]
    c = lax.axis_index("c")
    s = lax.axis_index("s")
    w = s * NC + c
    arow = pl.multiple_of(s * RPW, 8)
    pltpu.sync_copy(zeros_hbm, acc.at[pl.ds(arow, RPW)])
    plsc.subcore_barrier()

    def stage_idx(bi, cidx):
        base = pl.multiple_of((w + cidx * NW) * CHUNK, 8)
        pltpu.async_copy(src_hbm.at[pl.ds(base, CHUNK)], sib[bi], isem[bi])
        pltpu.async_copy(dst_hbm.at[pl.ds(base, CHUNK)], dib[bi], isem[bi])

    def wait_idx(bi):
        pltpu.make_async_copy(src_hbm.at[pl.ds(0, CHUNK)], sib[bi],
                              isem[bi]).wait()
        pltpu.make_async_copy(dst_hbm.at[pl.ds(0, CHUNK)], dib[bi],
                              isem[bi]).wait()

    def start_gather(bi, br):
        pltpu.async_copy(g_hbm.at[sib[bi]], rows[br], gsem[br])

    def wait_gather(bi, br):
        pltpu.make_async_copy(g_hbm.at[sib[bi]], rows[br], gsem[br]).wait()

    for bi in range(NIB):
        stage_idx(bi, bi)
    for bi in range(2):
        wait_idx(bi)
        start_gather(bi, bi)

    def group(t, carry):
        for b in range(NIB):
            cidx = t * NIB + b
            br = b % NRB
            wait_gather(b, br)
            pltpu.sync_copy(rows[br], acc.at[dib[b]], add=True)

            @pl.when(cidx + NIB < FULL_TRIPS)
            def _():
                stage_idx(b, cidx + NIB)

            @pl.when(cidx + 2 < FULL_TRIPS)
            def _():
                bi2 = (b + 2) % NIB
                wait_idx(bi2)
                start_gather(bi2, br)
        return carry

    lax.fori_loop(0, FULL_TRIPS // NIB, group, 0)

    for b in range(FULL_TRIPS % NIB):
        wait_gather(b, b % NRB)
        pltpu.sync_copy(rows[b % NRB], acc.at[dib[b]], add=True)

    @pl.when(w < REM)
    def _():
        base = pl.multiple_of((FULL_TRIPS * NW + w) * CHUNK, 8)
        pltpu.sync_copy(src_hbm.at[pl.ds(base, CHUNK)], sib[0])
        pltpu.sync_copy(dst_hbm.at[pl.ds(base, CHUNK)], dib[0])
        pltpu.async_copy(g_hbm.at[sib[0]], rows[0], gsem[0]).wait()
        pltpu.sync_copy(rows[0], acc.at[dib[0]], add=True)

    plsc.subcore_barrier()
    orow = pl.multiple_of(c * NPA + s * RPW, 8)
    pltpu.sync_copy(acc.at[pl.ds(arow, RPW)], out_hbm.at[pl.ds(orow, RPW)])


ROWS_BLK = 2000
NBLK = N // ROWS_BLK


def _tca_body(degp0, degp1, x, W1, dis_ref, g1_ref):
    deg = degp0[:, 0:1] + degp1[:, 0:1] + 1.0
    dis = lax.rsqrt(deg)
    dis_ref[...] = dis
    g1_ref[...] = jnp.dot(x[...], W1[...],
                          preferred_element_type=jnp.float32) * dis


def _tca(degp0, degp1, x, W1):
    return pl.pallas_call(
        _tca_body,
        grid=(NBLK,),
        in_specs=[
            pl.BlockSpec((ROWS_BLK, H), lambda i: (i, 0)),
            pl.BlockSpec((ROWS_BLK, H), lambda i: (i, 0)),
            pl.BlockSpec((ROWS_BLK, H), lambda i: (i, 0)),
            pl.BlockSpec((H, H), lambda i: (0, 0)),
        ],
        out_specs=[
            pl.BlockSpec((ROWS_BLK, 1), lambda i: (i, 0)),
            pl.BlockSpec((ROWS_BLK, H), lambda i: (i, 0)),
        ],
        out_shape=[
            jax.ShapeDtypeStruct((N, 1), jnp.float32),
            jax.ShapeDtypeStruct((N, H), jnp.float32),
        ],
    )(degp0, degp1, x, W1)


def _tcb_body(sp0, sp1, g, dis, b, W, gn_ref):
    a = jnp.maximum(dis[...] * (sp0[...] + sp1[...] + g[...]) + b[...], 0.0)
    gn_ref[...] = dis[...] * jnp.dot(a, W[...],
                                     preferred_element_type=jnp.float32)


def _tcb(sp0, sp1, g, dis, b, W):
    return pl.pallas_call(
        _tcb_body,
        grid=(NBLK,),
        in_specs=[
            pl.BlockSpec((ROWS_BLK, H), lambda i: (i, 0)),
            pl.BlockSpec((ROWS_BLK, H), lambda i: (i, 0)),
            pl.BlockSpec((ROWS_BLK, H), lambda i: (i, 0)),
            pl.BlockSpec((ROWS_BLK, 1), lambda i: (i, 0)),
            pl.BlockSpec((1, H), lambda i: (0, 0)),
            pl.BlockSpec((H, H), lambda i: (0, 0)),
        ],
        out_specs=pl.BlockSpec((ROWS_BLK, H), lambda i: (i, 0)),
        out_shape=jax.ShapeDtypeStruct((N, H), jnp.float32),
    )(sp0, sp1, g, dis, b, W)


def _tcc_body(sp0, sp1, g, dis, b, batchb, Wl, bl, out_ref, sums, counts):
    i = pl.program_id(0)

    @pl.when(i == 0)
    def _():
        sums[...] = jnp.zeros_like(sums)
        counts[...] = jnp.zeros_like(counts)

    a = dis[...] * (sp0[...] + sp1[...] + g[...]) + b[...]
    gid = lax.broadcasted_iota(jnp.int32, (ROWS_BLK, G), 1)
    m = (batchb[...] == gid).astype(jnp.float32)
    sums[...] += lax.dot_general(m, a, (((0,), (0,)), ((), ())),
                                 preferred_element_type=jnp.float32)
    counts[...] += lax.dot_general(
        m, jnp.ones((ROWS_BLK, H), jnp.float32), (((0,), (0,)), ((), ())),
        preferred_element_type=jnp.float32)

    @pl.when(i == NBLK - 1)
    def _():
        pooled = sums[...] / jnp.maximum(counts[...], 1.0)
        logits = jnp.dot(pooled, Wl[...],
                         preferred_element_type=jnp.float32) + bl[...]
        z = logits - jnp.max(logits, axis=-1, keepdims=True)
        e = jnp.exp(z)
        out_ref[...] = e / jnp.sum(e, axis=-1, keepdims=True)


def _tcc(sp0, sp1, g, dis, b, batch2d, Wl, bl):
    return pl.pallas_call(
        _tcc_body,
        grid=(NBLK,),
        in_specs=[
            pl.BlockSpec((ROWS_BLK, H), lambda i: (i, 0)),
            pl.BlockSpec((ROWS_BLK, H), lambda i: (i, 0)),
            pl.BlockSpec((ROWS_BLK, H), lambda i: (i, 0)),
            pl.BlockSpec((ROWS_BLK, 1), lambda i: (i, 0)),
            pl.BlockSpec((1, H), lambda i: (0, 0)),
            pl.BlockSpec((ROWS_BLK, 1), lambda i: (i, 0)),
            pl.BlockSpec((H, C), lambda i: (0, 0)),
            pl.BlockSpec((1, C), lambda i: (0, 0)),
        ],
        out_specs=pl.BlockSpec((G, C), lambda i: (0, 0)),
        out_shape=jax.ShapeDtypeStruct((G, C), jnp.float32),
        scratch_shapes=[
            pltpu.VMEM((G, H), jnp.float32),
            pltpu.VMEM((G, H), jnp.float32),
        ],
    )(sp0, sp1, g, dis, b, batch2d, Wl, bl)


def kernel(x, edge_index, batch, W1, b1, W2, b2, W3, b3, Wl, bl):
    src1d = edge_index[0]
    dst1d = edge_index[1]
    batch2d = batch.reshape(N, 1)
    zeros_spmm = jnp.zeros((RPW, H), jnp.float32)
    ones_rows = jnp.ones((CHUNK, H), jnp.float32)
    b1r, b2r, b3r = b1.reshape(1, H), b2.reshape(1, H), b3.reshape(1, H)
    blr = bl.reshape(1, C)

    degp = _deg_kernel(dst1d, ones_rows, zeros_spmm)
    dis, g1 = _tca(degp[:N], degp[NPA:NPA + N], x, W1)
    s1 = _spmm_kernel(src1d, dst1d, g1, zeros_spmm)
    g2 = _tcb(s1[:N], s1[NPA:NPA + N], g1, dis, b1r, W2)
    s2 = _spmm_kernel(src1d, dst1d, g2, zeros_spmm)
    g3 = _tcb(s2[:N], s2[NPA:NPA + N], g2, dis, b2r, W3)
    s3 = _spmm_kernel(src1d, dst1d, g3, zeros_spmm)
    return _tcc(s3[:N], s3[NPA:NPA + N], g3, dis, b3r, batch2d, Wl, blr)
